# baseline (speedup 1.0000x reference)
import functools

import jax
import jax.numpy as jnp
from jax import lax
from jax.experimental import pallas as pl
from jax.experimental.pallas import tpu as pltpu
from jax.experimental.pallas import tpu_sc as plsc

DIM = 128
NC = 2
NS = 16
NW = NC * NS
CHUNK = 128
NBUF = 3
GROUPS = DIM // 32
INTER = 4


def _sc_gather(num3, pe_b16, pe_f32, nchunk):
    total = NW * nchunk * CHUNK
    ngroups = nchunk // NBUF
    tail = nchunk - ngroups * NBUF
    assert tail < NBUF
    for j in range(ngroups * NBUF, nchunk):
        assert j % INTER != 0
    mesh = plsc.VectorSubcoreMesh(core_axis_name="c", subcore_axis_name="s")

    scratch = (
        [pltpu.VMEM((nchunk, CHUNK), jnp.int32)]
        + [pltpu.VMEM((CHUNK, DIM // 2), jnp.int32) for _ in range(NBUF)]
        + [pltpu.VMEM((CHUNK, DIM), jnp.int32) for _ in range(NBUF)]
        + [pltpu.SemaphoreType.DMA for _ in range(2 * NBUF)]
    )

    @functools.partial(
        pl.kernel,
        mesh=mesh,
        out_type=jax.ShapeDtypeStruct((total, DIM), jnp.int32),
        scratch_types=scratch,
        compiler_params=pltpu.CompilerParams(use_tc_tiling_on_sc=False),
    )
    def k(idx_hbm, peb_hbm, pe32_hbm, out_hbm, *refs):
        idx_v = refs[0]
        gb = refs[1:1 + NBUF]
        fb = refs[1 + NBUF:1 + 2 * NBUF]
        sem_g = refs[1 + 2 * NBUF:1 + 3 * NBUF]
        sem_o = refs[1 + 3 * NBUF:1 + 4 * NBUF]

        wid = lax.axis_index("s") * NC + lax.axis_index("c")
        base = wid * (nchunk * CHUNK)
        pltpu.sync_copy(idx_hbm.at[wid], idx_v)

        mask = jnp.int32(-65536)

        def expand(src, dst):
            @plsc.parallel_loop(0, CHUNK, unroll=4)
            def row(r):
                for g in range(GROUPS):
                    v = src[r, pl.ds(16 * g, 16)]
                    dst[r, pl.ds(32 * g, 16)] = v << 16
                    dst[r, pl.ds(32 * g + 16, 16)] = v & mask

        def wait_g16(b):
            pltpu.make_async_copy(
                peb_hbm.at[pl.ds(0, CHUNK)], gb[b], sem_g[b]
            ).wait()

        def wait_g32(b):
            pltpu.make_async_copy(
                pe32_hbm.at[pl.ds(0, CHUNK)], fb[b], sem_g[b]
            ).wait()

        def wait_o(b):
            pltpu.make_async_copy(
                fb[b], out_hbm.at[pl.ds(base, CHUNK)], sem_o[b]
            ).wait()

        for b in range(NBUF):
            if b % INTER == 0:
                pltpu.async_copy(pe32_hbm.at[idx_v.at[b]], fb[b], sem_g[b])
            else:
                pltpu.async_copy(peb_hbm.at[idx_v.at[b]], gb[b], sem_g[b])

        def group(g, carry):
            for b in range(NBUF):
                j = g * NBUF + b
                direct = j % INTER == 0

                @pl.when(direct)
                def _():
                    wait_g32(b)

                @pl.when(jnp.logical_not(direct))
                def _():
                    wait_g16(b)

                    @pl.when(g > 0)
                    def _():
                        wait_o(b)
                    expand(gb[b], fb[b])

                pltpu.async_copy(
                    fb[b], out_hbm.at[pl.ds(base + j * CHUNK, CHUNK)],
                    sem_o[b],
                )

                jn = j + NBUF

                @pl.when(jn < nchunk)
                def _():
                    directn = jn % INTER == 0

                    @pl.when(directn)
                    def _():
                        wait_o(b)
                        pltpu.async_copy(
                            pe32_hbm.at[idx_v.at[jn]], fb[b], sem_g[b]
                        )

                    @pl.when(jnp.logical_not(directn))
                    def _():
                        pltpu.async_copy(
                            peb_hbm.at[idx_v.at[jn]], gb[b], sem_g[b]
                        )
            return carry

        lax.fori_loop(0, ngroups, group, 0)

        for b in range(tail):
            j = ngroups * NBUF + b
            wait_g16(b)
            wait_o(b)
            expand(gb[b], fb[b])
            pltpu.async_copy(
                fb[b], out_hbm.at[pl.ds(base + j * CHUNK, CHUNK)], sem_o[b]
            )

        for b in range(NBUF):
            wait_o(b)

    return k(num3, pe_b16, pe_f32)


def kernel(num, pe):
    batch, hist = num.shape
    total = batch * hist
    nrows, dim = pe.shape
    nchunk = total // (NW * CHUNK)
    num3 = num.reshape(NW, nchunk, CHUNK).astype(jnp.int32)
    pe_sw = (
        pe.astype(jnp.bfloat16)
        .reshape(nrows, dim // 32, 2, 16)
        .transpose(0, 1, 3, 2)
        .reshape(nrows, dim // 2, 2)
    )
    pe_b16 = jax.lax.bitcast_convert_type(pe_sw, jnp.int32)
    pe_f32 = jax.lax.bitcast_convert_type(pe, jnp.int32)
    out = _sc_gather(num3, pe_b16, pe_f32, nchunk)
    return jax.lax.bitcast_convert_type(out, jnp.float32).reshape(
        batch, hist, DIM)


# speedup vs baseline: 1.6791x; 1.6791x over previous
import functools

import jax
import jax.numpy as jnp
from jax import lax
from jax.experimental import pallas as pl
from jax.experimental.pallas import tpu as pltpu
from jax.experimental.pallas import tpu_sc as plsc

DIM = 128
NC = 2
NS = 16
NW = NC * NS
CHUNK = 128
NBUF = 5


def _sc_gather(num3, pe, nchunk):
    total = NW * nchunk * CHUNK
    ngroups = nchunk // NBUF
    mesh = plsc.VectorSubcoreMesh(core_axis_name="c", subcore_axis_name="s")

    scratch = (
        [pltpu.VMEM((nchunk, CHUNK), jnp.int32)]
        + [pltpu.VMEM((CHUNK, DIM), jnp.float32) for _ in range(NBUF)]
        + [pltpu.SemaphoreType.DMA for _ in range(2 * NBUF)]
    )

    @functools.partial(
        pl.kernel,
        mesh=mesh,
        out_type=jax.ShapeDtypeStruct((total, DIM), jnp.float32),
        scratch_types=scratch,
    )
    def k(idx_hbm, pe_hbm, out_hbm, *refs):
        idx_v = refs[0]
        rows = refs[1:1 + NBUF]
        sem_g = refs[1 + NBUF:1 + 2 * NBUF]
        sem_o = refs[1 + 2 * NBUF:1 + 3 * NBUF]

        wid = lax.axis_index("s") * NC + lax.axis_index("c")
        base = wid * (nchunk * CHUNK)
        pltpu.sync_copy(idx_hbm.at[wid], idx_v)

        for b in range(NBUF):
            pltpu.async_copy(pe_hbm.at[idx_v.at[b]], rows[b], sem_g[b])

        def group(g, carry):
            for b in range(NBUF):
                j = g * NBUF + b
                pltpu.make_async_copy(
                    pe_hbm.at[pl.ds(0, CHUNK)], rows[b], sem_g[b]
                ).wait()
                pltpu.async_copy(
                    rows[b], out_hbm.at[pl.ds(base + j * CHUNK, CHUNK)],
                    sem_o[b],
                )
            @pl.when(g + 1 < ngroups)
            def _():
                for b in range(NBUF):
                    jn = (g + 1) * NBUF + b
                    pltpu.make_async_copy(
                        rows[b], out_hbm.at[pl.ds(base, CHUNK)], sem_o[b]
                    ).wait()
                    pltpu.async_copy(
                        pe_hbm.at[idx_v.at[jn]], rows[b], sem_g[b]
                    )
            return carry

        lax.fori_loop(0, ngroups, group, 0)

        for b in range(NBUF):
            pltpu.make_async_copy(
                rows[b], out_hbm.at[pl.ds(base, CHUNK)], sem_o[b]
            ).wait()

        for j in range(ngroups * NBUF, nchunk):
            pltpu.async_copy(pe_hbm.at[idx_v.at[j]], rows[0], sem_g[0]).wait()
            pltpu.sync_copy(rows[0], out_hbm.at[pl.ds(base + j * CHUNK, CHUNK)])

    return k(num3, pe)


def kernel(num, pe):
    batch, hist = num.shape
    total = batch * hist
    nchunk = total // (NW * CHUNK)
    num3 = num.reshape(NW, nchunk, CHUNK).astype(jnp.int32)
    out = _sc_gather(num3, pe, nchunk)
    return out.reshape(batch, hist, DIM)
